# baseline (speedup 1.0000x reference)
import jax
import jax.numpy as jnp
from jax import lax
from jax.experimental import pallas as pl
from jax.experimental.pallas import tpu as pltpu
from jax.experimental.pallas import tpu_sc as plsc

B = 64
A = 100000
L = 512
LANES = 16
NUM_CORES = 2
NUM_SUBCORES = 16
NW = NUM_CORES * NUM_SUBCORES
RPW = B // NW
CHUNK = 128
NCH = L // CHUNK
KCH = RPW * NCH
ZN = 50000
NZ = A // ZN


def _red_scalar(vec, op):
    acc = vec[0]
    for i in range(1, LANES):
        acc = op(acc, vec[i])
    return acc


def _body(logits_hbm, legal_hbm, out_hbm, idx_v, vals_v, zbuf_v,
          zsem, gsem, ssem, isem):
    wid = lax.axis_index("s") * NUM_CORES + lax.axis_index("c")
    row0 = wid * RPW

    idx_cp = pltpu.make_async_copy(legal_hbm.at[wid], idx_v, isem)
    idx_cp.start()

    zvec = jnp.zeros((LANES,), jnp.float32)
    for j in range(ZN // LANES):
        zbuf_v[pl.ds(j * LANES, LANES)] = zvec

    zcps = []
    for r in range(RPW):
        for z in range(NZ):
            off = pl.multiple_of((row0 + r) * A + z * ZN, 8)
            cp = pltpu.make_async_copy(zbuf_v, out_hbm.at[pl.ds(off, ZN)], zsem)
            cp.start()
            zcps.append(cp)

    idx_cp.wait()

    for k in range(KCH):
        base = (row0 + k // NCH) * A
        for i in range(CHUNK // LANES):
            sl = idx_v[k, pl.ds(i * LANES, LANES)]
            idx_v[k, pl.ds(i * LANES, LANES)] = sl + base

    gcps = []
    for k in range(KCH):
        cp = pltpu.make_async_copy(logits_hbm.at[idx_v.at[k]], vals_v.at[k], gsem)
        cp.start()
        gcps.append(cp)
    for cp in gcps:
        cp.wait()

    for r in range(RPW):
        ks = range(r * NCH, (r + 1) * NCH)
        m = None
        for k in ks:
            for i in range(CHUNK // LANES):
                sl = vals_v[k, pl.ds(i * LANES, LANES)]
                m = sl if m is None else jnp.maximum(m, sl)
        mx = _red_scalar(m, jnp.maximum)
        s = jnp.zeros((LANES,), jnp.float32)
        for k in ks:
            for i in range(CHUNK // LANES):
                e = jnp.exp(vals_v[k, pl.ds(i * LANES, LANES)] - mx)
                vals_v[k, pl.ds(i * LANES, LANES)] = e
                s = s + e
        tot = _red_scalar(s, jnp.add)
        for k in ks:
            for i in range(CHUNK // LANES):
                vals_v[k, pl.ds(i * LANES, LANES)] = (
                    vals_v[k, pl.ds(i * LANES, LANES)] / tot)

    for cp in zcps:
        cp.wait()

    scps = []
    for k in range(KCH):
        cp = pltpu.make_async_copy(vals_v.at[k], out_hbm.at[idx_v.at[k]], ssem)
        cp.start()
        scps.append(cp)
    for cp in scps:
        cp.wait()


def kernel(logits, legal_actions):
    mesh = plsc.VectorSubcoreMesh(core_axis_name="c", subcore_axis_name="s")
    run = pl.kernel(
        _body,
        mesh=mesh,
        out_type=jax.ShapeDtypeStruct((B * A,), jnp.float32),
        scratch_types=[
            pltpu.VMEM((KCH, CHUNK), jnp.int32),
            pltpu.VMEM((KCH, CHUNK), jnp.float32),
            pltpu.VMEM((ZN,), jnp.float32),
            pltpu.SemaphoreType.DMA,
            pltpu.SemaphoreType.DMA,
            pltpu.SemaphoreType.DMA,
            pltpu.SemaphoreType.DMA,
        ],
    )
    out = run(logits.reshape(B * A), legal_actions.reshape(NW, KCH, CHUNK))
    return out.reshape(B, A)


# speedup vs baseline: 1.0620x; 1.0620x over previous
import jax
import jax.numpy as jnp
from jax import lax
from jax.experimental import pallas as pl
from jax.experimental.pallas import tpu as pltpu
from jax.experimental.pallas import tpu_sc as plsc

B = 64
A = 100000
L = 512
LANES = 16
NUM_CORES = 2
NUM_SUBCORES = 16
NW = NUM_CORES * NUM_SUBCORES
RPW = B // NW
CHUNK = 128
NCH = L // CHUNK
KCH = RPW * NCH


def _red_scalar(vec, op):
    acc = vec[0]
    for i in range(1, LANES):
        acc = op(acc, vec[i])
    return acc


def _body(logits_hbm, legal_hbm, out_hbm, idx_v, fidx_v, vals_v, row_v,
          gsem, ssem, isem):
    wid = lax.axis_index("s") * NUM_CORES + lax.axis_index("c")
    row0 = wid * RPW

    idx_cp = pltpu.make_async_copy(legal_hbm.at[wid], idx_v, isem)
    idx_cp.start()

    zvec = jnp.zeros((LANES,), jnp.float32)

    def _zero_step(j, carry):
        row_v[pl.ds(pl.multiple_of(j * LANES, LANES), LANES)] = zvec
        return carry

    lax.fori_loop(0, A // LANES, _zero_step, 0)

    idx_cp.wait()

    for k in range(KCH):
        base = (row0 + k // NCH) * A
        for i in range(CHUNK // LANES):
            sl = idx_v[k, pl.ds(i * LANES, LANES)]
            fidx_v[k, pl.ds(i * LANES, LANES)] = sl + base

    gcps = []
    for k in range(KCH):
        cp = pltpu.make_async_copy(logits_hbm.at[fidx_v.at[k]], vals_v.at[k], gsem)
        cp.start()
        gcps.append(cp)
    for cp in gcps:
        cp.wait()

    for r in range(RPW):
        ks = range(r * NCH, (r + 1) * NCH)
        m = None
        for k in ks:
            for i in range(CHUNK // LANES):
                sl = vals_v[k, pl.ds(i * LANES, LANES)]
                m = sl if m is None else jnp.maximum(m, sl)
        mx = _red_scalar(m, jnp.maximum)
        s = jnp.zeros((LANES,), jnp.float32)
        for k in ks:
            for i in range(CHUNK // LANES):
                e = jnp.exp(vals_v[k, pl.ds(i * LANES, LANES)] - mx)
                vals_v[k, pl.ds(i * LANES, LANES)] = e
                s = s + e
        tot = _red_scalar(s, jnp.add)
        for k in ks:
            for i in range(CHUNK // LANES):
                vals_v[k, pl.ds(i * LANES, LANES)] = (
                    vals_v[k, pl.ds(i * LANES, LANES)] / tot)

    for r in range(RPW):
        ks = range(r * NCH, (r + 1) * NCH)
        for k in ks:
            for i in range(CHUNK // LANES):
                plsc.store_scatter(row_v, [idx_v[k, pl.ds(i * LANES, LANES)]],
                                   vals_v[k, pl.ds(i * LANES, LANES)])
        off = pl.multiple_of((row0 + r) * A, 8)
        cp = pltpu.make_async_copy(row_v, out_hbm.at[pl.ds(off, A)], ssem)
        cp.start()
        cp.wait()
        if r + 1 < RPW:
            for k in ks:
                for i in range(CHUNK // LANES):
                    plsc.store_scatter(row_v,
                                       [idx_v[k, pl.ds(i * LANES, LANES)]], zvec)


def kernel(logits, legal_actions):
    mesh = plsc.VectorSubcoreMesh(core_axis_name="c", subcore_axis_name="s")
    run = pl.kernel(
        _body,
        mesh=mesh,
        compiler_params=pltpu.CompilerParams(needs_layout_passes=False),
        out_type=jax.ShapeDtypeStruct((B * A,), jnp.float32),
        scratch_types=[
            pltpu.VMEM((KCH, CHUNK), jnp.int32),
            pltpu.VMEM((KCH, CHUNK), jnp.int32),
            pltpu.VMEM((KCH, CHUNK), jnp.float32),
            pltpu.VMEM((A,), jnp.float32),
            pltpu.SemaphoreType.DMA,
            pltpu.SemaphoreType.DMA,
            pltpu.SemaphoreType.DMA,
        ],
    )
    out = run(logits.reshape(B * A), legal_actions.reshape(NW, KCH, CHUNK))
    return out.reshape(B, A)


# speedup vs baseline: 1.3105x; 1.2339x over previous
import jax
import jax.numpy as jnp
from jax import lax
from jax.experimental import pallas as pl
from jax.experimental.pallas import tpu as pltpu
from jax.experimental.pallas import tpu_sc as plsc

B = 64
A = 100000
L = 512
LANES = 16
NUM_CORES = 2
NUM_SUBCORES = 16
NW = NUM_CORES * NUM_SUBCORES
RPW = B // NW
CHUNK = 128
NCH = L // CHUNK
KCH = RPW * NCH


def _red_scalar(vec, op):
    acc = vec[0]
    for i in range(1, LANES):
        acc = op(acc, vec[i])
    return acc


def _body(logits_hbm, legal_hbm, out_hbm, idx_v, fidx_v, vals_v, row_v,
          gsem, ssem, isem):
    wid = lax.axis_index("s") * NUM_CORES + lax.axis_index("c")
    row0 = wid * RPW

    idx_cp = pltpu.make_async_copy(legal_hbm.at[wid], idx_v, isem)
    idx_cp.start()

    zvec = jnp.zeros((LANES,), jnp.float32)

    FILL_UNROLL = 25

    def _zero_step(j, carry):
        base = pl.multiple_of(j * (FILL_UNROLL * LANES), LANES)
        for u in range(FILL_UNROLL):
            row_v[pl.ds(base + u * LANES, LANES)] = zvec
        return carry

    lax.fori_loop(0, A // (FILL_UNROLL * LANES), _zero_step, 0)

    idx_cp.wait()

    for k in range(KCH):
        base = (row0 + k // NCH) * A
        for i in range(CHUNK // LANES):
            sl = idx_v[k, pl.ds(i * LANES, LANES)]
            fidx_v[k, pl.ds(i * LANES, LANES)] = sl + base

    gcps = []
    for k in range(KCH):
        cp = pltpu.make_async_copy(logits_hbm.at[fidx_v.at[k]], vals_v.at[k], gsem)
        cp.start()
        gcps.append(cp)
    for cp in gcps:
        cp.wait()

    for r in range(RPW):
        ks = range(r * NCH, (r + 1) * NCH)
        m = None
        for k in ks:
            for i in range(CHUNK // LANES):
                sl = vals_v[k, pl.ds(i * LANES, LANES)]
                m = sl if m is None else jnp.maximum(m, sl)
        mx = _red_scalar(m, jnp.maximum)
        s = jnp.zeros((LANES,), jnp.float32)
        for k in ks:
            for i in range(CHUNK // LANES):
                e = jnp.exp(vals_v[k, pl.ds(i * LANES, LANES)] - mx)
                vals_v[k, pl.ds(i * LANES, LANES)] = e
                s = s + e
        tot = _red_scalar(s, jnp.add)
        for k in ks:
            for i in range(CHUNK // LANES):
                vals_v[k, pl.ds(i * LANES, LANES)] = (
                    vals_v[k, pl.ds(i * LANES, LANES)] / tot)

    for r in range(RPW):
        ks = range(r * NCH, (r + 1) * NCH)
        for k in ks:
            for i in range(CHUNK // LANES):
                plsc.store_scatter(row_v, [idx_v[k, pl.ds(i * LANES, LANES)]],
                                   vals_v[k, pl.ds(i * LANES, LANES)])
        off = pl.multiple_of((row0 + r) * A, 8)
        cp = pltpu.make_async_copy(row_v, out_hbm.at[pl.ds(off, A)], ssem)
        cp.start()
        cp.wait()
        if r + 1 < RPW:
            for k in ks:
                for i in range(CHUNK // LANES):
                    plsc.store_scatter(row_v,
                                       [idx_v[k, pl.ds(i * LANES, LANES)]], zvec)


def kernel(logits, legal_actions):
    mesh = plsc.VectorSubcoreMesh(core_axis_name="c", subcore_axis_name="s")
    run = pl.kernel(
        _body,
        mesh=mesh,
        compiler_params=pltpu.CompilerParams(needs_layout_passes=False),
        out_type=jax.ShapeDtypeStruct((B * A,), jnp.float32),
        scratch_types=[
            pltpu.VMEM((KCH, CHUNK), jnp.int32),
            pltpu.VMEM((KCH, CHUNK), jnp.int32),
            pltpu.VMEM((KCH, CHUNK), jnp.float32),
            pltpu.VMEM((A,), jnp.float32),
            pltpu.SemaphoreType.DMA,
            pltpu.SemaphoreType.DMA,
            pltpu.SemaphoreType.DMA,
        ],
    )
    out = run(logits.reshape(B * A), legal_actions.reshape(NW, KCH, CHUNK))
    return out.reshape(B, A)


# speedup vs baseline: 1.9998x; 1.5260x over previous
import jax
import jax.numpy as jnp
from jax import lax
from jax.experimental import pallas as pl
from jax.experimental.pallas import tpu as pltpu
from jax.experimental.pallas import tpu_sc as plsc

B = 64
A = 100000
L = 512
LANES = 16
NUM_CORES = 2
NUM_SUBCORES = 16
NW = NUM_CORES * NUM_SUBCORES
RPW = B // NW
CHUNK = 128
NCH = L // CHUNK
KCH = RPW * NCH
T = (A + 127) // 128
AP = T * 128
G = B // 8


def _red_scalar(vec, op):
    acc = vec[0]
    for i in range(1, LANES):
        acc = op(acc, vec[i])
    return acc


def _body(logits_hbm, legal_hbm, out_hbm, idx_v, fidx_v, vals_v, row_v,
          gsem, ssem, isem):
    wid = lax.axis_index("s") * NUM_CORES + lax.axis_index("c")
    row0 = wid * RPW

    idx_cp = pltpu.make_async_copy(legal_hbm.at[wid], idx_v, isem)
    idx_cp.start()

    zvec = jnp.zeros((LANES,), jnp.float32)

    def _zero_step(j, carry):
        for p in range(2):
            t = j * 2 + p
            for u in range(128 // LANES):
                row_v[t, pl.ds(u * LANES, LANES)] = zvec
        return carry

    lax.fori_loop(0, T // 2, _zero_step, 0)

    idx_cp.wait()

    for k in range(KCH):
        base = (row0 + k // NCH) * A
        for i in range(CHUNK // LANES):
            sl = idx_v[k, pl.ds(i * LANES, LANES)]
            fidx_v[k, pl.ds(i * LANES, LANES)] = sl + base

    gcps = []
    for k in range(KCH):
        cp = pltpu.make_async_copy(logits_hbm.at[fidx_v.at[k]], vals_v.at[k], gsem)
        cp.start()
        gcps.append(cp)
    for cp in gcps:
        cp.wait()

    for r in range(RPW):
        ks = range(r * NCH, (r + 1) * NCH)
        m = None
        for k in ks:
            for i in range(CHUNK // LANES):
                sl = vals_v[k, pl.ds(i * LANES, LANES)]
                m = sl if m is None else jnp.maximum(m, sl)
        mx = _red_scalar(m, jnp.maximum)
        s = jnp.zeros((LANES,), jnp.float32)
        for k in ks:
            for i in range(CHUNK // LANES):
                e = jnp.exp(vals_v[k, pl.ds(i * LANES, LANES)] - mx)
                vals_v[k, pl.ds(i * LANES, LANES)] = e
                s = s + e
        tot = _red_scalar(s, jnp.add)
        for k in ks:
            for i in range(CHUNK // LANES):
                vals_v[k, pl.ds(i * LANES, LANES)] = (
                    vals_v[k, pl.ds(i * LANES, LANES)] / tot)

    for r in range(RPW):
        row = row0 + r
        g = row >> 3
        rr = row & 7
        ks = range(r * NCH, (r + 1) * NCH)
        for k in ks:
            for i in range(CHUNK // LANES):
                ci = idx_v[k, pl.ds(i * LANES, LANES)]
                plsc.store_scatter(row_v, [ci >> 7, ci & 127],
                                   vals_v[k, pl.ds(i * LANES, LANES)])
        cp = pltpu.make_async_copy(row_v, out_hbm.at[g, :, rr, :], ssem)
        cp.start()
        cp.wait()
        if r + 1 < RPW:
            for k in ks:
                for i in range(CHUNK // LANES):
                    ci = idx_v[k, pl.ds(i * LANES, LANES)]
                    plsc.store_scatter(row_v, [ci >> 7, ci & 127], zvec)


def kernel(logits, legal_actions):
    mesh = plsc.VectorSubcoreMesh(core_axis_name="c", subcore_axis_name="s")
    run = pl.kernel(
        _body,
        mesh=mesh,
        compiler_params=pltpu.CompilerParams(needs_layout_passes=False),
        out_type=jax.ShapeDtypeStruct((G, T, 8, 128), jnp.float32),
        scratch_types=[
            pltpu.VMEM((KCH, CHUNK), jnp.int32),
            pltpu.VMEM((KCH, CHUNK), jnp.int32),
            pltpu.VMEM((KCH, CHUNK), jnp.float32),
            pltpu.VMEM((T, 128), jnp.float32),
            pltpu.SemaphoreType.DMA,
            pltpu.SemaphoreType.DMA,
            pltpu.SemaphoreType.DMA,
        ],
    )
    out4 = run(logits.reshape(B * A), legal_actions.reshape(NW, KCH, CHUNK))
    out = out4.transpose(0, 2, 1, 3).reshape(B, AP)[:, :A]
    return out


# speedup vs baseline: 2.0397x; 1.0200x over previous
import jax
import jax.numpy as jnp
from jax import lax
from jax.experimental import pallas as pl
from jax.experimental.pallas import tpu as pltpu
from jax.experimental.pallas import tpu_sc as plsc

B = 64
A = 100000
L = 512
LANES = 16
NUM_CORES = 2
NUM_SUBCORES = 16
NW = NUM_CORES * NUM_SUBCORES
RPW = B // NW
CHUNK = 128
NCH = L // CHUNK
KCH = RPW * NCH
T = (A + 127) // 128
AP = T * 128
G = B // 8


def _red_scalar(vec, op):
    acc = vec[0]
    for i in range(1, LANES):
        acc = op(acc, vec[i])
    return acc


def _body(logits_hbm, legal_hbm, out_hbm, idx_v, fidx_v, vals_v, row_v,
          gsem, ssem, isem):
    wid = lax.axis_index("s") * NUM_CORES + lax.axis_index("c")
    row0 = wid * RPW

    idx_cp = pltpu.make_async_copy(legal_hbm.at[wid], idx_v, isem)
    idx_cp.start()

    idx_cp.wait()

    for k in range(KCH):
        base = (row0 + k // NCH) * A
        for i in range(CHUNK // LANES):
            sl = idx_v[k, pl.ds(i * LANES, LANES)]
            fidx_v[k, pl.ds(i * LANES, LANES)] = sl + base

    gcps = []
    for k in range(KCH):
        cp = pltpu.make_async_copy(logits_hbm.at[fidx_v.at[k]], vals_v.at[k], gsem)
        cp.start()
        gcps.append(cp)

    zvec = jnp.zeros((LANES,), jnp.float32)

    def _zero_step(j, carry):
        for p in range(2):
            t = j * 2 + p
            for u in range(128 // LANES):
                row_v[t, pl.ds(u * LANES, LANES)] = zvec
        return carry

    lax.fori_loop(0, T // 2, _zero_step, 0)

    for cp in gcps:
        cp.wait()

    for r in range(RPW):
        ks = range(r * NCH, (r + 1) * NCH)
        m = None
        for k in ks:
            for i in range(CHUNK // LANES):
                sl = vals_v[k, pl.ds(i * LANES, LANES)]
                m = sl if m is None else jnp.maximum(m, sl)
        mx = _red_scalar(m, jnp.maximum)
        s = jnp.zeros((LANES,), jnp.float32)
        for k in ks:
            for i in range(CHUNK // LANES):
                e = jnp.exp(vals_v[k, pl.ds(i * LANES, LANES)] - mx)
                vals_v[k, pl.ds(i * LANES, LANES)] = e
                s = s + e
        tot = _red_scalar(s, jnp.add)
        for k in ks:
            for i in range(CHUNK // LANES):
                vals_v[k, pl.ds(i * LANES, LANES)] = (
                    vals_v[k, pl.ds(i * LANES, LANES)] / tot)

    for r in range(RPW):
        row = row0 + r
        g = row >> 3
        rr = row & 7
        ks = range(r * NCH, (r + 1) * NCH)
        for k in ks:
            for i in range(CHUNK // LANES):
                ci = idx_v[k, pl.ds(i * LANES, LANES)]
                plsc.store_scatter(row_v, [ci >> 7, ci & 127],
                                   vals_v[k, pl.ds(i * LANES, LANES)])
        cp = pltpu.make_async_copy(row_v, out_hbm.at[g, :, rr, :], ssem)
        cp.start()
        cp.wait()
        if r + 1 < RPW:
            for k in ks:
                for i in range(CHUNK // LANES):
                    ci = idx_v[k, pl.ds(i * LANES, LANES)]
                    plsc.store_scatter(row_v, [ci >> 7, ci & 127], zvec)


def kernel(logits, legal_actions):
    mesh = plsc.VectorSubcoreMesh(core_axis_name="c", subcore_axis_name="s")
    run = pl.kernel(
        _body,
        mesh=mesh,
        compiler_params=pltpu.CompilerParams(needs_layout_passes=False),
        out_type=jax.ShapeDtypeStruct((G, T, 8, 128), jnp.float32),
        scratch_types=[
            pltpu.VMEM((KCH, CHUNK), jnp.int32),
            pltpu.VMEM((KCH, CHUNK), jnp.int32),
            pltpu.VMEM((KCH, CHUNK), jnp.float32),
            pltpu.VMEM((T, 128), jnp.float32),
            pltpu.SemaphoreType.DMA,
            pltpu.SemaphoreType.DMA,
            pltpu.SemaphoreType.DMA,
        ],
    )
    out4 = run(logits.reshape(B * A), legal_actions.reshape(NW, KCH, CHUNK))
    out = out4.transpose(0, 2, 1, 3).reshape(B, AP)[:, :A]
    return out
